# baseline (speedup 1.0000x reference)
import functools

import jax
import jax.numpy as jnp
from jax import lax
from jax.experimental import pallas as pl
from jax.experimental.pallas import tpu as pltpu
from jax.experimental.pallas import tpu_sc as plsc

N = 10000
K = 32
D = 128
DW = D // 2
NLANES = 16
NGRP = DW // NLANES
NTILES = 32
NPT = 320
CN = 4
CE = CN * K


def _pack_tc(features):
    u = jax.lax.bitcast_convert_type(features, jnp.uint32)
    packed = (u[:, :DW] >> 16) | (u[:, DW:] & jnp.uint32(0xFFFF0000))
    return jax.lax.bitcast_convert_type(packed, jnp.int32)


def _aggsum_sc(neighbors_i32, feat_packed):
    mesh = plsc.VectorSubcoreMesh(core_axis_name="c", subcore_axis_name="s")

    @functools.partial(
        pl.kernel,
        out_type=pltpu.MemorySpace.HBM((N, D), jnp.float32),
        mesh=mesh,
        compiler_params=pltpu.CompilerParams(use_tc_tiling_on_sc=False),
        scratch_types=[
            pltpu.VMEM((NPT * K,), jnp.int32),
            pltpu.VMEM((CE, DW), jnp.int32),
            pltpu.VMEM((CE, DW), jnp.int32),
            pltpu.VMEM((2 * CN, D), jnp.float32),
            pltpu.VMEM((2 * CN, D), jnp.float32),
            pltpu.VMEM_SHARED((N, DW), jnp.int32),
            pltpu.SemaphoreType.DMA,
            pltpu.SemaphoreType.DMA,
            pltpu.SemaphoreType.DMA,
            pltpu.SemaphoreType.DMA,
        ],
    )
    def body(nbr_hbm, feat_hbm, out_hbm, idx_v, buf0, buf1, ob0, ob1, tab,
             sem0, sem1, osem0, osem1):
        sid = lax.axis_index("s")
        cid = lax.axis_index("c")
        wid = cid * 16 + sid
        base = wid * NPT
        nq = jnp.where(wid == NTILES - 1, 5, NPT // CN // 4)
        pltpu.sync_copy(feat_hbm.at[pl.ds(sid * 624, 624)], tab.at[pl.ds(sid * 624, 624)])

        @pl.when(sid == 0)
        def _():
            pltpu.sync_copy(feat_hbm.at[pl.ds(9984, 16)], tab.at[pl.ds(9984, 16)])

        @pl.when(wid < NTILES - 1)
        def _():
            pltpu.sync_copy(nbr_hbm.at[pl.ds(base * K, NPT * K)],
                            idx_v.at[pl.ds(0, NPT * K)])

        @pl.when(wid == NTILES - 1)
        def _():
            pltpu.sync_copy(nbr_hbm.at[pl.ds(base * K, 80 * K)],
                            idx_v.at[pl.ds(0, 80 * K)])
        plsc.subcore_barrier()

        lastc = nq * 4 - 1

        def start(ci, buf, sem):
            ci = jnp.minimum(ci, lastc)
            pltpu.async_copy(tab.at[idx_v.at[pl.ds(ci * CE, CE)]], buf, sem)

        def gwait(buf, sem):
            pltpu.make_async_copy(tab.at[pl.ds(0, CE)], buf, sem).wait()

        def owait(ob, osem):
            pltpu.make_async_copy(ob, out_hbm.at[pl.ds(0, 2 * CN)], osem).wait()

        def reduce_chunk(buf, ob, half):
            for n in range(CN):
                def halves(r, g):
                    w = buf[r, pl.ds(g * NLANES, NLANES)]
                    lo = jax.lax.bitcast_convert_type(w << 16, jnp.float32)
                    hi = jax.lax.bitcast_convert_type(w, jnp.float32)
                    return lo, hi

                accs = []
                for g in range(NGRP):
                    a, b = halves(n * K, g)
                    accs.extend([a, b])
                for k in range(1, K):
                    for g in range(NGRP):
                        a, b = halves(n * K + k, g)
                        accs[2 * g] = accs[2 * g] + a
                        accs[2 * g + 1] = accs[2 * g + 1] + b
                row = half * CN + n
                for g in range(NGRP):
                    ob[row, pl.ds(g * NLANES, NLANES)] = accs[2 * g]
                    ob[row, pl.ds(DW + g * NLANES, NLANES)] = accs[2 * g + 1]

        start(0, buf0, sem0)
        start(1, buf1, sem1)

        def quad(q, carry):
            @pl.when(q > 0)
            def _():
                owait(ob0, osem0)

            gwait(buf0, sem0)
            reduce_chunk(buf0, ob0, 0)
            start(4 * q + 2, buf0, sem0)
            gwait(buf1, sem1)
            reduce_chunk(buf1, ob0, 1)
            start(4 * q + 3, buf1, sem1)
            pltpu.async_copy(ob0, out_hbm.at[pl.ds(base + q * 4 * CN, 2 * CN)], osem0)

            @pl.when(q > 0)
            def _():
                owait(ob1, osem1)

            gwait(buf0, sem0)
            reduce_chunk(buf0, ob1, 0)

            @pl.when(q < nq - 1)
            def _():
                start(4 * q + 4, buf0, sem0)

            gwait(buf1, sem1)
            reduce_chunk(buf1, ob1, 1)

            @pl.when(q < nq - 1)
            def _():
                start(4 * q + 5, buf1, sem1)

            pltpu.async_copy(ob1, out_hbm.at[pl.ds(base + q * 4 * CN + 2 * CN, 2 * CN)], osem1)
            return carry

        lax.fori_loop(0, nq, quad, 0)
        owait(ob0, osem0)
        owait(ob1, osem1)

    return body(neighbors_i32, feat_packed)


def _lin1_body(feat, w1, bb, out):
    out[...] = (
        jnp.dot(feat[...], w1[...], preferred_element_type=jnp.float32) + bb[...]
    )


def _lin1_tc(features, W1, bb):
    BR = 2000
    return pl.pallas_call(
        _lin1_body,
        grid=(N // BR,),
        in_specs=[
            pl.BlockSpec((BR, D), lambda i: (i, 0)),
            pl.BlockSpec((D, D), lambda i: (0, 0)),
            pl.BlockSpec((1, D), lambda i: (0, 0)),
        ],
        out_specs=pl.BlockSpec((BR, D), lambda i: (i, 0)),
        out_shape=jax.ShapeDtypeStruct((N, D), jnp.float32),
    )(features, W1, bb)


def _lin2_body(part, aggs, wa, ba, w2, out):
    mean = (aggs[...] * (1.0 / K)).astype(jnp.bfloat16)
    a = (
        jnp.dot(mean, wa[...].astype(jnp.bfloat16),
                preferred_element_type=jnp.float32)
        + ba[...]
    )
    a = jnp.maximum(a, 0.0).astype(jnp.bfloat16)
    out[...] = part[...] + jnp.dot(
        a, w2[...].astype(jnp.bfloat16), preferred_element_type=jnp.float32
    )


def _lin2_tc(partial, aggsum, Wa, ba, W2):
    BR = 2000
    return pl.pallas_call(
        _lin2_body,
        grid=(N // BR,),
        in_specs=[
            pl.BlockSpec((BR, D), lambda i: (i, 0)),
            pl.BlockSpec((BR, D), lambda i: (i, 0)),
            pl.BlockSpec((D, D), lambda i: (0, 0)),
            pl.BlockSpec((1, D), lambda i: (0, 0)),
            pl.BlockSpec((D, D), lambda i: (0, 0)),
        ],
        out_specs=pl.BlockSpec((BR, D), lambda i: (i, 0)),
        out_shape=jax.ShapeDtypeStruct((N, D), jnp.float32),
    )(partial, aggsum, Wa, ba, W2)


def kernel(features, neighbors, W_agg, b_agg, W, b):
    nbr = neighbors.astype(jnp.int32).reshape(N * K)
    feat_packed = _pack_tc(features)
    aggsum = _aggsum_sc(nbr, feat_packed)
    partial = _lin1_tc(features, W[:D], b.reshape(1, D))
    return _lin2_tc(partial, aggsum, W_agg, b_agg.reshape(1, D), W[D:])


# speedup vs baseline: 1.4068x; 1.4068x over previous
import functools

import jax
import jax.numpy as jnp
from jax import lax
from jax.experimental import pallas as pl
from jax.experimental.pallas import tpu as pltpu
from jax.experimental.pallas import tpu_sc as plsc

N = 10000
K = 32
D = 128
DW = D // 2
NLANES = 16
NGRP = DW // NLANES
NTILES = 32
NPT = 320
CN = 4
CE = CN * K


def _pack_tc(features):
    u = jax.lax.bitcast_convert_type(features, jnp.uint32)
    packed = (u[:, :DW] >> 16) | (u[:, DW:] & jnp.uint32(0xFFFF0000))
    return jax.lax.bitcast_convert_type(packed, jnp.int32)


def _aggsum_sc(neighbors_i32, feat_packed):
    mesh = plsc.VectorSubcoreMesh(core_axis_name="c", subcore_axis_name="s")

    @functools.partial(
        pl.kernel,
        out_type=pltpu.MemorySpace.HBM((N, D), jnp.float32),
        mesh=mesh,
        compiler_params=pltpu.CompilerParams(use_tc_tiling_on_sc=False),
        scratch_types=[
            pltpu.VMEM((NPT * K,), jnp.int32),
            pltpu.VMEM((CE, DW), jnp.int32),
            pltpu.VMEM((CE, DW), jnp.int32),
            pltpu.VMEM((2 * CN, D), jnp.float32),
            pltpu.VMEM((2 * CN, D), jnp.float32),
            pltpu.VMEM_SHARED((N, DW), jnp.int32),
            pltpu.SemaphoreType.DMA,
            pltpu.SemaphoreType.DMA,
            pltpu.SemaphoreType.DMA,
            pltpu.SemaphoreType.DMA,
        ],
    )
    def body(nbr_hbm, feat_hbm, out_hbm, idx_v, buf0, buf1, ob0, ob1, tab,
             sem0, sem1, osem0, osem1):
        sid = lax.axis_index("s")
        cid = lax.axis_index("c")
        wid = cid * 16 + sid
        base = wid * NPT
        nq = jnp.where(wid == NTILES - 1, 5, NPT // CN // 4)
        pltpu.sync_copy(feat_hbm.at[pl.ds(sid * 624, 624)], tab.at[pl.ds(sid * 624, 624)])

        @pl.when(sid == 0)
        def _():
            pltpu.sync_copy(feat_hbm.at[pl.ds(9984, 16)], tab.at[pl.ds(9984, 16)])

        @pl.when(wid < NTILES - 1)
        def _():
            pltpu.sync_copy(nbr_hbm.at[pl.ds(base * K, NPT * K)],
                            idx_v.at[pl.ds(0, NPT * K)])

        @pl.when(wid == NTILES - 1)
        def _():
            pltpu.sync_copy(nbr_hbm.at[pl.ds(base * K, 80 * K)],
                            idx_v.at[pl.ds(0, 80 * K)])
        plsc.subcore_barrier()

        lastc = nq * 4 - 1

        def start(ci, buf, sem):
            ci = jnp.minimum(ci, lastc)
            pltpu.async_copy(tab.at[idx_v.at[pl.ds(ci * CE, CE)]], buf, sem)

        def gwait(buf, sem):
            pltpu.make_async_copy(tab.at[pl.ds(0, CE)], buf, sem).wait()

        def owait(ob, osem):
            pltpu.make_async_copy(ob, out_hbm.at[pl.ds(0, 2 * CN)], osem).wait()

        def reduce_chunk(buf, ob, half):
            def nbody(n, c):
                def halves(r, g):
                    w = buf[r, pl.ds(g * NLANES, NLANES)]
                    lo = jax.lax.bitcast_convert_type(w << 16, jnp.float32)
                    hi = jax.lax.bitcast_convert_type(w, jnp.float32)
                    return lo, hi

                accs = []
                for g in range(NGRP):
                    a, b = halves(n * K, g)
                    accs.extend([a, b])
                for k in range(1, K):
                    for g in range(NGRP):
                        a, b = halves(n * K + k, g)
                        accs[2 * g] = accs[2 * g] + a
                        accs[2 * g + 1] = accs[2 * g + 1] + b
                row = half * CN + n
                for g in range(NGRP):
                    ob[row, pl.ds(g * NLANES, NLANES)] = accs[2 * g]
                    ob[row, pl.ds(DW + g * NLANES, NLANES)] = accs[2 * g + 1]
                return c

            lax.fori_loop(0, CN, nbody, 0)

        start(0, buf0, sem0)
        start(1, buf1, sem1)

        def quad(q, carry):
            @pl.when(q > 0)
            def _():
                owait(ob0, osem0)

            gwait(buf0, sem0)
            reduce_chunk(buf0, ob0, 0)
            start(4 * q + 2, buf0, sem0)
            gwait(buf1, sem1)
            reduce_chunk(buf1, ob0, 1)
            start(4 * q + 3, buf1, sem1)
            pltpu.async_copy(ob0, out_hbm.at[pl.ds(base + q * 4 * CN, 2 * CN)], osem0)

            @pl.when(q > 0)
            def _():
                owait(ob1, osem1)

            gwait(buf0, sem0)
            reduce_chunk(buf0, ob1, 0)

            @pl.when(q < nq - 1)
            def _():
                start(4 * q + 4, buf0, sem0)

            gwait(buf1, sem1)
            reduce_chunk(buf1, ob1, 1)

            @pl.when(q < nq - 1)
            def _():
                start(4 * q + 5, buf1, sem1)

            pltpu.async_copy(ob1, out_hbm.at[pl.ds(base + q * 4 * CN + 2 * CN, 2 * CN)], osem1)
            return carry

        lax.fori_loop(0, nq, quad, 0)
        owait(ob0, osem0)
        owait(ob1, osem1)

    return body(neighbors_i32, feat_packed)


def _lin1_body(feat, w1, bb, out):
    out[...] = (
        jnp.dot(feat[...], w1[...], preferred_element_type=jnp.float32) + bb[...]
    )


def _lin1_tc(features, W1, bb):
    BR = 2000
    return pl.pallas_call(
        _lin1_body,
        grid=(N // BR,),
        in_specs=[
            pl.BlockSpec((BR, D), lambda i: (i, 0)),
            pl.BlockSpec((D, D), lambda i: (0, 0)),
            pl.BlockSpec((1, D), lambda i: (0, 0)),
        ],
        out_specs=pl.BlockSpec((BR, D), lambda i: (i, 0)),
        out_shape=jax.ShapeDtypeStruct((N, D), jnp.float32),
    )(features, W1, bb)


def _lin2_body(part, aggs, wa, ba, w2, out):
    mean = (aggs[...] * (1.0 / K)).astype(jnp.bfloat16)
    a = (
        jnp.dot(mean, wa[...].astype(jnp.bfloat16),
                preferred_element_type=jnp.float32)
        + ba[...]
    )
    a = jnp.maximum(a, 0.0).astype(jnp.bfloat16)
    out[...] = part[...] + jnp.dot(
        a, w2[...].astype(jnp.bfloat16), preferred_element_type=jnp.float32
    )


def _lin2_tc(partial, aggsum, Wa, ba, W2):
    BR = 2000
    return pl.pallas_call(
        _lin2_body,
        grid=(N // BR,),
        in_specs=[
            pl.BlockSpec((BR, D), lambda i: (i, 0)),
            pl.BlockSpec((BR, D), lambda i: (i, 0)),
            pl.BlockSpec((D, D), lambda i: (0, 0)),
            pl.BlockSpec((1, D), lambda i: (0, 0)),
            pl.BlockSpec((D, D), lambda i: (0, 0)),
        ],
        out_specs=pl.BlockSpec((BR, D), lambda i: (i, 0)),
        out_shape=jax.ShapeDtypeStruct((N, D), jnp.float32),
    )(partial, aggsum, Wa, ba, W2)


def kernel(features, neighbors, W_agg, b_agg, W, b):
    nbr = neighbors.astype(jnp.int32).reshape(N * K)
    feat_packed = _pack_tc(features)
    aggsum = _aggsum_sc(nbr, feat_packed)
    partial = _lin1_tc(features, W[:D], b.reshape(1, D))
    return _lin2_tc(partial, aggsum, W_agg, b_agg.reshape(1, D), W[D:])
